# baseline (speedup 1.0000x reference)
import functools
import math

import jax
import jax.numpy as jnp
from jax import lax
from jax.experimental import pallas as pl
from jax.experimental.pallas import tpu as pltpu
from jax.experimental.pallas import tpu_sc as plsc

VOCAB = 1000000
D_MODEL = 64
SCALE = math.sqrt(D_MODEL)

NUM_CORES = 2
NUM_SUBCORES = 16
NW = NUM_CORES * NUM_SUBCORES
LANES = 16

B_TOTAL = 4096 * 200
PER_W = B_TOTAL // NW
CHUNK = 800
NCHUNK = PER_W // CHUNK


def _emb_body(x_hbm, table_hbm, out_hbm, idx_v, rows_v, sem):
    wid = lax.axis_index("s") * NUM_CORES + lax.axis_index("c")
    base = wid * PER_W

    @pl.loop(0, NCHUNK)
    def _chunk(g):
        start = base + g * CHUNK
        pltpu.sync_copy(x_hbm.at[pl.ds(start, CHUNK)], idx_v)
        pltpu.async_copy(table_hbm.at[idx_v], rows_v, sem).wait()

        @pl.loop(0, CHUNK)
        def _scale(r):
            for j in range(D_MODEL // LANES):
                sl = pl.ds(j * LANES, LANES)
                rows_v[r, sl] = rows_v[r, sl] * SCALE

        pltpu.sync_copy(rows_v, out_hbm.at[pl.ds(start, CHUNK)])


@jax.jit
def _emb_call(x_flat, table):
    mesh = plsc.VectorSubcoreMesh(core_axis_name="c", subcore_axis_name="s")
    return pl.kernel(
        _emb_body,
        out_type=jax.ShapeDtypeStruct((B_TOTAL, D_MODEL), jnp.float32),
        mesh=mesh,
        scratch_types=[
            pltpu.VMEM((CHUNK,), jnp.int32),
            pltpu.VMEM((CHUNK, D_MODEL), jnp.float32),
            pltpu.SemaphoreType.DMA,
        ],
        compiler_params=pltpu.CompilerParams(use_tc_tiling_on_sc=False),
    )(x_flat, table)


def kernel(x, table):
    out = _emb_call(x.reshape(-1).astype(jnp.int32), table)
    return out.reshape(x.shape + (D_MODEL,))


# speedup vs baseline: 1.0912x; 1.0912x over previous
import math

import jax
import jax.numpy as jnp
from jax import lax
from jax.experimental import pallas as pl
from jax.experimental.pallas import tpu as pltpu
from jax.experimental.pallas import tpu_sc as plsc

VOCAB = 1000000
D_MODEL = 64
SCALE = math.sqrt(D_MODEL)

NUM_CORES = 2
NUM_SUBCORES = 16
NW = NUM_CORES * NUM_SUBCORES
LANES = 16

B_TOTAL = 4096 * 200
PER_W = B_TOTAL // NW
CHUNK = 800
NCHUNK = PER_W // CHUNK


def _emb_body(x_hbm, table_hbm, out_hbm,
              idx0, idx1, rows0, rows1, gsem0, gsem1, osem0, osem1):
    wid = lax.axis_index("s") * NUM_CORES + lax.axis_index("c")
    base = wid * PER_W

    idx = (idx0, idx1)
    rows = (rows0, rows1)
    gsem = (gsem0, gsem1)
    osem = (osem0, osem1)

    def scale_buf(buf):
        @pl.loop(0, CHUNK)
        def _scale(r):
            for j in range(D_MODEL // LANES):
                sl = pl.ds(j * LANES, LANES)
                buf[r, sl] = buf[r, sl] * SCALE

    pltpu.sync_copy(x_hbm.at[pl.ds(base, CHUNK)], idx[0])
    pltpu.async_copy(table_hbm.at[idx[0]], rows[0], gsem[0])

    @pl.loop(0, NCHUNK, step=2)
    def _pair(g):
        for ph in range(2):
            cur, nxt = ph, 1 - ph
            gg = g + ph

            @pl.when(gg + 1 < NCHUNK)
            def _prefetch():
                pltpu.sync_copy(
                    x_hbm.at[pl.ds(base + (gg + 1) * CHUNK, CHUNK)], idx[nxt])

                @pl.when(gg >= 1)
                def _drain_prev():
                    pltpu.make_async_copy(
                        rows[nxt],
                        out_hbm.at[pl.ds(base + (gg - 1) * CHUNK, CHUNK)],
                        osem[nxt]).wait()

                pltpu.async_copy(table_hbm.at[idx[nxt]], rows[nxt], gsem[nxt])

            pltpu.make_async_copy(
                table_hbm.at[idx[cur]], rows[cur], gsem[cur]).wait()
            scale_buf(rows[cur])
            pltpu.async_copy(
                rows[cur],
                out_hbm.at[pl.ds(base + gg * CHUNK, CHUNK)], osem[cur])

    pltpu.make_async_copy(
        rows[0], out_hbm.at[pl.ds(base + (NCHUNK - 2) * CHUNK, CHUNK)],
        osem[0]).wait()
    pltpu.make_async_copy(
        rows[1], out_hbm.at[pl.ds(base + (NCHUNK - 1) * CHUNK, CHUNK)],
        osem[1]).wait()


@jax.jit
def _emb_call(x_flat, table):
    mesh = plsc.VectorSubcoreMesh(core_axis_name="c", subcore_axis_name="s")
    return pl.kernel(
        _emb_body,
        out_type=jax.ShapeDtypeStruct((B_TOTAL, D_MODEL), jnp.float32),
        mesh=mesh,
        scratch_types=[
            pltpu.VMEM((CHUNK,), jnp.int32),
            pltpu.VMEM((CHUNK,), jnp.int32),
            pltpu.VMEM((CHUNK, D_MODEL), jnp.float32),
            pltpu.VMEM((CHUNK, D_MODEL), jnp.float32),
            pltpu.SemaphoreType.DMA,
            pltpu.SemaphoreType.DMA,
            pltpu.SemaphoreType.DMA,
            pltpu.SemaphoreType.DMA,
        ],
        compiler_params=pltpu.CompilerParams(use_tc_tiling_on_sc=False),
    )(x_flat, table)


def kernel(x, table):
    out = _emb_call(x.reshape(-1).astype(jnp.int32), table)
    return out.reshape(x.shape + (D_MODEL,))
